# baseline (speedup 1.0000x reference)
import functools

import jax
import jax.numpy as jnp
from jax import lax
from jax.experimental import pallas as pl
from jax.experimental.pallas import tpu as pltpu
from jax.experimental.pallas import tpu_sc as plsc

E = 8
D = 1024
FF = 2048
S = 2048
BT = 64
NPAD = S + E * BT
RCHUNK = 256
NW = 32
TPW = S // NW
FSPLIT = 2
FH = FF // FSPLIT


def _dot_t(a, b):
    return lax.dot_general(a, b, (((1,), (1,)), ((), ())),
                           preferred_element_type=jnp.float32)


def _router_body(x_ref, gate_ref, probs_ref, pos_ref, meta_ref, oh_ref):
    logits = _dot_t(x_ref[...], gate_ref[...])
    probs_ref[...] = jax.nn.softmax(logits, axis=-1)

    m = jnp.max(logits, axis=1, keepdims=True)
    eiota = lax.broadcasted_iota(jnp.int32, (S, E), 1)
    sel = jnp.min(jnp.where(logits >= m, eiota, E), axis=1)
    onehot = (sel[:, None] == eiota).astype(jnp.float32)
    oh_ref[...] = onehot

    counts = jnp.sum(onehot, axis=0, keepdims=True)
    segpad = jnp.ceil(counts / BT) * BT
    tri8 = (lax.broadcasted_iota(jnp.int32, (E, E), 0)
            < lax.broadcasted_iota(jnp.int32, (E, E), 1)).astype(jnp.float32)
    starts = jnp.dot(segpad, tri8, preferred_element_type=jnp.float32)

    tril = (lax.broadcasted_iota(jnp.int32, (RCHUNK, RCHUNK), 0)
            > lax.broadcasted_iota(jnp.int32, (RCHUNK, RCHUNK), 1)
            ).astype(jnp.float32)

    def body(c, base):
        oc = oh_ref[pl.ds(c * RCHUNK, RCHUNK), :]
        ranks = jnp.dot(tril, oc, preferred_element_type=jnp.float32) + base
        dest = jnp.sum((ranks + starts) * oc, axis=1)
        pos_ref[pl.ds(c * RCHUNK, RCHUNK)] = dest.astype(jnp.int32)
        return base + jnp.sum(oc, axis=0, keepdims=True)

    lax.fori_loop(0, S // RCHUNK, body, jnp.zeros((1, E), jnp.float32))

    nck = segpad / BT
    row0 = jnp.pad(starts, ((0, 0), (0, 8))).astype(jnp.int32)
    row1 = jnp.pad(nck, ((0, 0), (0, 8))).astype(jnp.int32)
    meta_ref[...] = jnp.concatenate([row0, row1], axis=0)


def _router(x, gate_w):
    return pl.pallas_call(
        _router_body,
        in_specs=[
            pl.BlockSpec((S, D), lambda: (0, 0)),
            pl.BlockSpec((E, D), lambda: (0, 0)),
        ],
        out_specs=[
            pl.BlockSpec((S, E), lambda: (0, 0)),
            pl.BlockSpec((S,), lambda: (0,)),
            pl.BlockSpec((2, 16), lambda: (0, 0)),
        ],
        out_shape=[
            jax.ShapeDtypeStruct((S, E), jnp.float32),
            jax.ShapeDtypeStruct((S,), jnp.int32),
            jax.ShapeDtypeStruct((2, 16), jnp.int32),
        ],
        scratch_shapes=[pltpu.VMEM((S, E), jnp.float32)],
    )(x, gate_w)


@functools.lru_cache(maxsize=None)
def _dispatch_kernel():
    mesh = plsc.VectorSubcoreMesh(core_axis_name="c", subcore_axis_name="s")

    @functools.partial(
        pl.kernel, mesh=mesh,
        out_type=jax.ShapeDtypeStruct((NPAD, D), jnp.float32),
        scratch_types=[
            pltpu.VMEM((TPW,), jnp.int32),
            pltpu.VMEM((TPW, D), jnp.float32),
            pltpu.SemaphoreType.DMA,
        ],
    )
    def dispatch(x_hbm, pos_hbm, xs_hbm, idx_v, rows_v, sem):
        wid = lax.axis_index("s") * 2 + lax.axis_index("c")
        base = wid * TPW
        pltpu.sync_copy(pos_hbm.at[pl.ds(base, TPW)], idx_v)
        pltpu.sync_copy(x_hbm.at[pl.ds(base, TPW)], rows_v)
        pltpu.async_copy(rows_v, xs_hbm.at[idx_v], sem).wait()

    return dispatch


def _dispatch(x, pos):
    return _dispatch_kernel()(x, pos)


def _experts_body(meta_ref, xs_ref, w1_ref, w3_ref, w2_ref, out_ref):
    e = pl.program_id(0)
    f = pl.program_id(1)
    start = pl.multiple_of(meta_ref[0, e], BT)
    nck = meta_ref[1, e]
    w1 = w1_ref[0].astype(jnp.bfloat16)
    w3 = w3_ref[0].astype(jnp.bfloat16)
    w2 = w2_ref[0].astype(jnp.bfloat16)

    def body(j, _):
        off = pl.multiple_of(start + j * BT, BT)
        xs = xs_ref[pl.ds(off, BT), :].astype(jnp.bfloat16)
        h = jax.nn.silu(_dot_t(xs, w1)) * _dot_t(xs, w3)
        out = _dot_t(h.astype(jnp.bfloat16), w2)
        prev = jnp.where(f == 0, 0.0, out_ref[pl.ds(off, BT), :])
        out_ref[pl.ds(off, BT), :] = prev + out
        return 0

    lax.fori_loop(0, nck, body, 0)


def _experts(xs, w1, w3, w2, meta):
    grid_spec = pltpu.PrefetchScalarGridSpec(
        num_scalar_prefetch=1,
        grid=(E, FSPLIT),
        in_specs=[
            pl.BlockSpec((NPAD, D), lambda e, f, m: (0, 0)),
            pl.BlockSpec((1, FH, D), lambda e, f, m: (e, f, 0)),
            pl.BlockSpec((1, FH, D), lambda e, f, m: (e, f, 0)),
            pl.BlockSpec((1, D, FH), lambda e, f, m: (e, 0, f)),
        ],
        out_specs=pl.BlockSpec((NPAD, D), lambda e, f, m: (0, 0)),
    )
    return pl.pallas_call(
        _experts_body,
        grid_spec=grid_spec,
        out_shape=jax.ShapeDtypeStruct((NPAD, D), jnp.float32),
    )(meta, xs, w1, w3, w2)


@functools.lru_cache(maxsize=None)
def _combine_kernel():
    mesh = plsc.VectorSubcoreMesh(core_axis_name="c", subcore_axis_name="s")

    @functools.partial(
        pl.kernel, mesh=mesh,
        out_type=jax.ShapeDtypeStruct((S, D), jnp.float32),
        scratch_types=[
            pltpu.VMEM((TPW,), jnp.int32),
            pltpu.VMEM((TPW, D), jnp.float32),
            pltpu.SemaphoreType.DMA,
        ],
    )
    def combine(outs_hbm, pos_hbm, final_hbm, idx_v, rows_v, sem):
        wid = lax.axis_index("s") * 2 + lax.axis_index("c")
        base = wid * TPW
        pltpu.sync_copy(pos_hbm.at[pl.ds(base, TPW)], idx_v)
        pltpu.async_copy(outs_hbm.at[idx_v], rows_v, sem).wait()
        pltpu.sync_copy(rows_v, final_hbm.at[pl.ds(base, TPW)])

    return combine


def _combine(outs, pos):
    return _combine_kernel()(outs, pos)


def kernel(hidden_states, gate_w, w1, w3, w2):
    x = hidden_states.reshape(S, D)
    probs, pos, meta = _router(x, gate_w)
    xs = _dispatch(x, pos)
    outs = _experts(xs, w1, w3, w2, meta)
    final = _combine(outs, pos)
    return final.reshape(hidden_states.shape), probs


# speedup vs baseline: 1.2527x; 1.2527x over previous
import functools

import jax
import jax.numpy as jnp
from jax import lax
from jax.experimental import pallas as pl
from jax.experimental.pallas import tpu as pltpu
from jax.experimental.pallas import tpu_sc as plsc

E = 8
D = 1024
FF = 2048
S = 2048
BT = 128
NPAD = S + E * BT
RCHUNK = 256
NW = 32
TPW = S // NW
FSPLIT = 4
FH = FF // FSPLIT


def _dot_t(a, b):
    return lax.dot_general(a, b, (((1,), (1,)), ((), ())),
                           preferred_element_type=jnp.float32)


def _router_body(x_ref, gate_ref, probs_ref, pos_ref, meta_ref, oh_ref):
    logits = _dot_t(x_ref[...], gate_ref[...])
    probs_ref[...] = jax.nn.softmax(logits, axis=-1)

    m = jnp.max(logits, axis=1, keepdims=True)
    eiota = lax.broadcasted_iota(jnp.int32, (S, E), 1)
    sel = jnp.min(jnp.where(logits >= m, eiota, E), axis=1)
    onehot = (sel[:, None] == eiota).astype(jnp.float32)
    oh_ref[...] = onehot

    counts = jnp.sum(onehot, axis=0, keepdims=True)
    segpad = jnp.ceil(counts / BT) * BT
    tri8 = (lax.broadcasted_iota(jnp.int32, (E, E), 0)
            < lax.broadcasted_iota(jnp.int32, (E, E), 1)).astype(jnp.float32)
    starts = jnp.dot(segpad, tri8, preferred_element_type=jnp.float32)

    tril = (lax.broadcasted_iota(jnp.int32, (RCHUNK, RCHUNK), 0)
            > lax.broadcasted_iota(jnp.int32, (RCHUNK, RCHUNK), 1)
            ).astype(jnp.float32)

    def body(c, base):
        oc = oh_ref[pl.ds(c * RCHUNK, RCHUNK), :]
        ranks = jnp.dot(tril, oc, preferred_element_type=jnp.float32) + base
        dest = jnp.sum((ranks + starts) * oc, axis=1)
        pos_ref[pl.ds(c * RCHUNK, RCHUNK)] = dest.astype(jnp.int32)
        return base + jnp.sum(oc, axis=0, keepdims=True)

    lax.fori_loop(0, S // RCHUNK, body, jnp.zeros((1, E), jnp.float32))

    nck = segpad / BT
    row0 = jnp.pad(starts, ((0, 0), (0, 8))).astype(jnp.int32)
    row1 = jnp.pad(nck, ((0, 0), (0, 8))).astype(jnp.int32)
    meta_ref[...] = jnp.concatenate([row0, row1], axis=0)


def _router(x, gate_w):
    return pl.pallas_call(
        _router_body,
        in_specs=[
            pl.BlockSpec((S, D), lambda: (0, 0)),
            pl.BlockSpec((E, D), lambda: (0, 0)),
        ],
        out_specs=[
            pl.BlockSpec((S, E), lambda: (0, 0)),
            pl.BlockSpec((S,), lambda: (0,)),
            pl.BlockSpec((2, 16), lambda: (0, 0)),
        ],
        out_shape=[
            jax.ShapeDtypeStruct((S, E), jnp.float32),
            jax.ShapeDtypeStruct((S,), jnp.int32),
            jax.ShapeDtypeStruct((2, 16), jnp.int32),
        ],
        scratch_shapes=[pltpu.VMEM((S, E), jnp.float32)],
    )(x, gate_w)


@functools.lru_cache(maxsize=None)
def _dispatch_kernel():
    mesh = plsc.VectorSubcoreMesh(core_axis_name="c", subcore_axis_name="s")

    @functools.partial(
        pl.kernel, mesh=mesh,
        out_type=jax.ShapeDtypeStruct((NPAD, D), jnp.float32),
        scratch_types=[
            pltpu.VMEM((TPW,), jnp.int32),
            pltpu.VMEM((TPW, D), jnp.float32),
            pltpu.SemaphoreType.DMA,
        ],
    )
    def dispatch(x_hbm, pos_hbm, xs_hbm, idx_v, rows_v, sem):
        wid = lax.axis_index("s") * 2 + lax.axis_index("c")
        base = wid * TPW
        pltpu.sync_copy(pos_hbm.at[pl.ds(base, TPW)], idx_v)
        pltpu.sync_copy(x_hbm.at[pl.ds(base, TPW)], rows_v)
        pltpu.async_copy(rows_v, xs_hbm.at[idx_v], sem).wait()

    return dispatch


def _dispatch(x, pos):
    return _dispatch_kernel()(x, pos)


def _experts_body(meta_ref, xs_ref, w1_ref, w3_ref, w2_ref, out_ref):
    e = pl.program_id(0)
    f = pl.program_id(1)
    start = pl.multiple_of(meta_ref[0, e], BT)
    nck = meta_ref[1, e]
    w1 = w1_ref[0]
    w3 = w3_ref[0]
    w2 = w2_ref[0]

    def body(j, _):
        off = pl.multiple_of(start + j * BT, BT)
        xs = xs_ref[pl.ds(off, BT), :]
        h = jax.nn.silu(_dot_t(xs, w1)) * _dot_t(xs, w3)
        out = _dot_t(h, w2)
        prev = jnp.where(f == 0, 0.0, out_ref[pl.ds(off, BT), :])
        out_ref[pl.ds(off, BT), :] = prev + out
        return 0

    lax.fori_loop(0, nck, body, 0)


def _experts(xs, w1, w3, w2, meta):
    grid_spec = pltpu.PrefetchScalarGridSpec(
        num_scalar_prefetch=1,
        grid=(E, FSPLIT),
        in_specs=[
            pl.BlockSpec((NPAD, D), lambda e, f, m: (0, 0)),
            pl.BlockSpec((1, FH, D), lambda e, f, m: (e, f, 0)),
            pl.BlockSpec((1, FH, D), lambda e, f, m: (e, f, 0)),
            pl.BlockSpec((1, D, FH), lambda e, f, m: (e, 0, f)),
        ],
        out_specs=pl.BlockSpec((NPAD, D), lambda e, f, m: (0, 0)),
    )
    return pl.pallas_call(
        _experts_body,
        grid_spec=grid_spec,
        out_shape=jax.ShapeDtypeStruct((NPAD, D), jnp.float32),
    )(meta, xs, w1, w3, w2)


@functools.lru_cache(maxsize=None)
def _combine_kernel():
    mesh = plsc.VectorSubcoreMesh(core_axis_name="c", subcore_axis_name="s")

    @functools.partial(
        pl.kernel, mesh=mesh,
        out_type=jax.ShapeDtypeStruct((S, D), jnp.float32),
        scratch_types=[
            pltpu.VMEM((TPW,), jnp.int32),
            pltpu.VMEM((TPW, D), jnp.float32),
            pltpu.SemaphoreType.DMA,
        ],
    )
    def combine(outs_hbm, pos_hbm, final_hbm, idx_v, rows_v, sem):
        wid = lax.axis_index("s") * 2 + lax.axis_index("c")
        base = wid * TPW
        pltpu.sync_copy(pos_hbm.at[pl.ds(base, TPW)], idx_v)
        pltpu.async_copy(outs_hbm.at[idx_v], rows_v, sem).wait()
        pltpu.sync_copy(rows_v, final_hbm.at[pl.ds(base, TPW)])

    return combine


def _combine(outs, pos):
    return _combine_kernel()(outs, pos)


def kernel(hidden_states, gate_w, w1, w3, w2):
    x = hidden_states.reshape(S, D)
    probs, pos, meta = _router(x, gate_w)
    xs = _dispatch(x, pos)
    outs = _experts(xs, w1, w3, w2, meta)
    final = _combine(outs, pos)
    return final.reshape(hidden_states.shape), probs
